# baseline (speedup 1.0000x reference)
import jax
import jax.numpy as jnp
from jax.experimental import pallas as pl

_B, _S, _D, _E, _K = 4, 4096, 2048, 16, 2
_M = _B * _S
_NS = 2
_BM = 1024
_STEPS = _M // (_NS * _BM)
_MS = _M // _NS


def _top2(logits, w_out_ref, i_out_ref):
    m1 = jnp.max(logits, axis=-1)
    i1 = jnp.argmax(logits, axis=-1).astype(jnp.int32)
    lane = jax.lax.broadcasted_iota(jnp.int32, logits.shape, 1)
    masked = jnp.where(lane == i1[:, None], -jnp.inf, logits)
    m2 = jnp.max(masked, axis=-1)
    i2 = jnp.argmax(masked, axis=-1).astype(jnp.int32)
    e2 = jnp.exp(m2 - m1)
    denom = 1.0 + e2
    w_out_ref[...] = jnp.stack([1.0 / denom, e2 / denom], axis=-1)
    i_out_ref[...] = jnp.stack([i1, i2], axis=-1)


def _router_body(*refs):
    x_refs = refs[:_NS]
    wt_ref = refs[_NS]
    w_refs = refs[_NS + 1 : 2 * _NS + 1]
    i_refs = refs[2 * _NS + 1 : 3 * _NS + 1]
    l_refs = refs[3 * _NS + 1 :]
    wt = wt_ref[...]
    for j in range(_NS):
        logits = jnp.dot(x_refs[j][...], wt, preferred_element_type=jnp.float32)
        l_refs[j][...] = logits
        _top2(logits, w_refs[j], i_refs[j])


@jax.jit
def kernel(x, W):
    xm = x.reshape(_M, _D)
    wt = W.T

    def row_map(j):
        return lambda i: (i * _NS + j, 0)

    own_map = lambda i: (i, 0)

    in_specs = [pl.BlockSpec((_BM, _D), row_map(j)) for j in range(_NS)]
    in_specs.append(pl.BlockSpec((_D, _E), lambda i: (0, 0)))
    out_specs = (
        [pl.BlockSpec((_BM, _K), own_map) for _ in range(_NS)]
        + [pl.BlockSpec((_BM, _K), own_map) for _ in range(_NS)]
        + [pl.BlockSpec((_BM, _E), own_map) for _ in range(_NS)]
    )
    out_shape = (
        [jax.ShapeDtypeStruct((_MS, _K), jnp.float32) for _ in range(_NS)]
        + [jax.ShapeDtypeStruct((_MS, _K), jnp.int32) for _ in range(_NS)]
        + [jax.ShapeDtypeStruct((_MS, _E), jnp.float32) for _ in range(_NS)]
    )

    outs = pl.pallas_call(
        _router_body,
        grid=(_STEPS,),
        in_specs=in_specs,
        out_specs=out_specs,
        out_shape=out_shape,
    )(*([xm] * _NS), wt)

    w_full = _interleave(outs[:_NS], _K)
    i_full = _interleave(outs[_NS : 2 * _NS], _K)
    l_full = _interleave(outs[2 * _NS :], _E)

    return (
        w_full.reshape(_B, _S, _K),
        i_full.reshape(_B, _S, _K),
        l_full.reshape(_B, _S, _E),
    )


def _interleave(parts, width):
    stacked = jnp.stack([p.reshape(_STEPS, _BM, width) for p in parts], axis=1)
    return stacked.reshape(_M, width)


# speedup vs baseline: 1.0456x; 1.0456x over previous
import jax
import jax.numpy as jnp
from jax.experimental import pallas as pl
from jax.experimental.pallas import tpu as pltpu

_B, _S, _D, _E, _K = 4, 4096, 2048, 16, 2
_M = _B * _S
_BM = 1024


def _router_body(x_ref, wt_ref, w_out_ref, i_out_ref, logits_ref):
    logits = jnp.dot(x_ref[...], wt_ref[...], preferred_element_type=jnp.float32)
    logits_ref[...] = logits

    m1 = jnp.max(logits, axis=-1)
    i1 = jnp.argmax(logits, axis=-1).astype(jnp.int32)
    lane = jax.lax.broadcasted_iota(jnp.int32, logits.shape, 1)
    masked = jnp.where(lane == i1[:, None], -jnp.inf, logits)
    m2 = jnp.max(masked, axis=-1)
    i2 = jnp.argmax(masked, axis=-1).astype(jnp.int32)

    e2 = jnp.exp(m2 - m1)
    denom = 1.0 + e2
    w_out_ref[...] = jnp.stack([1.0 / denom, e2 / denom], axis=-1)
    i_out_ref[...] = jnp.stack([i1, i2], axis=-1)


@jax.jit
def kernel(x, W):
    xm = x.reshape(_M, _D)
    wt = W.T

    w_out, i_out, logits = pl.pallas_call(
        _router_body,
        grid=(_M // _BM,),
        in_specs=[
            pl.BlockSpec((_BM, _D), lambda i: (i, 0)),
            pl.BlockSpec((_D, _E), lambda i: (0, 0)),
        ],
        out_specs=[
            pl.BlockSpec((_BM, _K), lambda i: (i, 0)),
            pl.BlockSpec((_BM, _K), lambda i: (i, 0)),
            pl.BlockSpec((_BM, _E), lambda i: (i, 0)),
        ],
        out_shape=[
            jax.ShapeDtypeStruct((_M, _K), jnp.float32),
            jax.ShapeDtypeStruct((_M, _K), jnp.int32),
            jax.ShapeDtypeStruct((_M, _E), jnp.float32),
        ],
        compiler_params=pltpu.CompilerParams(
            dimension_semantics=("arbitrary",),
        ),
    )(xm, wt)

    return (
        w_out.reshape(_B, _S, _K),
        i_out.reshape(_B, _S, _K),
        logits.reshape(_B, _S, _E),
    )


# speedup vs baseline: 1.0529x; 1.0069x over previous
import jax
import jax.numpy as jnp
from jax.experimental import pallas as pl
from jax.experimental.pallas import tpu as pltpu

_B, _S, _D, _E, _K = 4, 4096, 2048, 16, 2
_M = _B * _S
_BM = 1024


def _router_body(x_ref, wt_ref, w_out_ref, i_out_ref, logits_ref):
    logits = jnp.dot(x_ref[...], wt_ref[...], preferred_element_type=jnp.float32)
    logits_ref[...] = logits

    m1 = jnp.max(logits, axis=-1)
    i1 = jnp.argmax(logits, axis=-1).astype(jnp.int32)
    lane = jax.lax.broadcasted_iota(jnp.int32, logits.shape, 1)
    masked = jnp.where(lane == i1[:, None], -jnp.inf, logits)
    m2 = jnp.max(masked, axis=-1)
    i2 = jnp.argmax(masked, axis=-1).astype(jnp.int32)

    e2 = jnp.exp(m2 - m1)
    denom = 1.0 + e2
    w_out_ref[...] = jnp.stack([1.0 / denom, e2 / denom], axis=-1)
    i_out_ref[...] = jnp.stack([i1, i2], axis=-1)


@jax.jit
def kernel(x, W):
    xm = x.reshape(_M, _D)
    wt = W.T

    w_out, i_out, logits = pl.pallas_call(
        _router_body,
        grid=(_M // _BM,),
        in_specs=[
            pl.BlockSpec((_BM, _D), lambda i: (i, 0)),
            pl.BlockSpec((_D, _E), lambda i: (0, 0)),
        ],
        out_specs=[
            pl.BlockSpec((_BM, _K), lambda i: (i, 0)),
            pl.BlockSpec((_BM, _K), lambda i: (i, 0)),
            pl.BlockSpec((_BM, _E), lambda i: (i, 0)),
        ],
        out_shape=[
            jax.ShapeDtypeStruct((_M, _K), jnp.float32),
            jax.ShapeDtypeStruct((_M, _K), jnp.int32),
            jax.ShapeDtypeStruct((_M, _E), jnp.float32),
        ],
        compiler_params=pltpu.CompilerParams(
            dimension_semantics=("parallel",),
        ),
    )(xm, wt)

    return (
        w_out.reshape(_B, _S, _K),
        i_out.reshape(_B, _S, _K),
        logits.reshape(_B, _S, _E),
    )
